# baseline (speedup 1.0000x reference)
import functools

import jax
import jax.numpy as jnp
from jax import lax
from jax.experimental import pallas as pl
from jax.experimental.pallas import tpu as pltpu
from jax.experimental.pallas import tpu_sc as plsc

BATCH = 16384
EMBED = 32
HID = 64
ACT = 6
PACK = 4
ROW = EMBED * PACK

NUM_CORES = 2
NUM_SUBCORES = 16
NUM_WORKERS = NUM_CORES * NUM_SUBCORES
B_PER_W = BATCH // NUM_WORKERS


def _sc_gather(table128, idx_hi):
    mesh = plsc.VectorSubcoreMesh(core_axis_name="c", subcore_axis_name="s")

    @functools.partial(
        pl.kernel,
        mesh=mesh,
        out_type=jax.ShapeDtypeStruct((BATCH, ROW), jnp.float32),
        scratch_types=[
            pltpu.VMEM((B_PER_W,), jnp.int32),
            pltpu.VMEM((B_PER_W, ROW), jnp.float32),
            pltpu.SemaphoreType.DMA,
        ],
    )
    def gather_kernel(idx_hbm, table_hbm, out_hbm, idx_v, rows_v, sem):
        wid = lax.axis_index("s") * NUM_CORES + lax.axis_index("c")
        base = wid * B_PER_W
        pltpu.sync_copy(idx_hbm.at[pl.ds(base, B_PER_W)], idx_v)
        pltpu.async_copy(table_hbm.at[idx_v], rows_v, sem).wait()
        pltpu.sync_copy(rows_v, out_hbm.at[pl.ds(base, B_PER_W)])

    return gather_kernel(idx_hi, table128)


def _mlp_body(x_ref, k_ref, w1_ref, b1_ref, w2_ref, b2_ref, o_ref):
    h = jnp.dot(x_ref[...], w1_ref[...], preferred_element_type=jnp.float32)
    h = jnp.maximum(h + b1_ref[...], 0.0)
    q4 = jnp.dot(h, w2_ref[...], preferred_element_type=jnp.float32)
    q4 = q4 + b2_ref[...]
    group = lax.broadcasted_iota(jnp.int32, q4.shape, 1) // ACT
    q4 = jnp.where(group == k_ref[...], q4, 0.0)
    o_ref[...] = (q4[:, 0:ACT] + q4[:, ACT:2 * ACT]
                  + q4[:, 2 * ACT:3 * ACT] + q4[:, 3 * ACT:4 * ACT])


def _tc_mlp(x, k, W1s, b1s, W2s, b2s):
    nblk = 8
    blk = BATCH // nblk
    return pl.pallas_call(
        _mlp_body,
        grid=(nblk,),
        in_specs=[
            pl.BlockSpec((blk, ROW), lambda i: (i, 0)),
            pl.BlockSpec((blk, 1), lambda i: (i, 0)),
            pl.BlockSpec((ROW, PACK * HID), lambda i: (0, 0)),
            pl.BlockSpec((1, PACK * HID), lambda i: (0, 0)),
            pl.BlockSpec((PACK * HID, PACK * ACT), lambda i: (0, 0)),
            pl.BlockSpec((1, PACK * ACT), lambda i: (0, 0)),
        ],
        out_specs=pl.BlockSpec((blk, ACT), lambda i: (i, 0)),
        out_shape=jax.ShapeDtypeStruct((BATCH, ACT), jnp.float32),
    )(x, k, W1s, b1s, W2s, b2s)


def kernel(state, table, W1, b1, W2, b2):
    state = state.astype(jnp.int32)
    table128 = table.reshape(table.shape[0] // PACK, ROW)
    x = _sc_gather(table128, state // PACK)
    k = (state % PACK).reshape(BATCH, 1)
    W1s = jax.scipy.linalg.block_diag(W1, W1, W1, W1)
    W2s = jax.scipy.linalg.block_diag(W2, W2, W2, W2)
    b1s = jnp.tile(b1, PACK).reshape(1, PACK * HID)
    b2s = jnp.tile(b2, PACK).reshape(1, PACK * ACT)
    return _tc_mlp(x, k, W1s, b1s, W2s, b2s)
